# Initial kernel scaffold; baseline (speedup 1.0000x reference)
import jax
import jax.numpy as jnp
from jax.experimental import pallas as pl


def kernel(hidden_states, gate_w, w_gate_up, w_down, ws_gate_up, ws_down):
    raise NotImplementedError("write your pallas kernel here")


# speedup vs baseline: 1.9402x; 1.9402x over previous
import functools

import jax
import jax.numpy as jnp
from jax import lax
from jax.experimental import pallas as pl
from jax.experimental.pallas import tpu as pltpu

_E = 8
_TOPK = 2
_D = 1024
_DFF = 1408
_T = 2048
_BLK = 256
_NTILES = _T * _TOPK // _BLK + _E
_CAP = _NTILES * _BLK


def _router_body(x_ref, gw_ref, pos1_ref, pos2_ref, w1_ref, w2_ref, te_ref):
    x = x_ref[...]
    gw = gw_ref[...]
    logits = lax.dot_general(x, gw, (((1,), (1,)), ((), ())),
                             preferred_element_type=jnp.float32)
    col = lax.broadcasted_iota(jnp.int32, (_T, _E), 1)
    m1 = jnp.max(logits, axis=1, keepdims=True)
    top1 = jnp.min(jnp.where(logits == m1, col, _E), axis=1, keepdims=True)
    oh1 = col == top1
    neg = jnp.float32(-3.4e38)
    l2 = jnp.where(oh1, neg, logits)
    m2 = jnp.max(l2, axis=1, keepdims=True)
    top2 = jnp.min(jnp.where(l2 == m2, col, _E), axis=1, keepdims=True)
    oh2 = col == top2
    w1 = jax.nn.sigmoid(m1 - m2)
    w1_ref[...] = w1
    w2_ref[...] = 1.0 - w1

    inc = oh1.astype(jnp.float32) + oh2.astype(jnp.float32)
    c = jnp.concatenate([jnp.zeros((1, _E), jnp.float32), inc[:-1]], axis=0)
    k = 1
    while k < _T:
        c = c + jnp.concatenate(
            [jnp.zeros((k, _E), jnp.float32), c[:-k]], axis=0)
        k *= 2
    counts = jnp.sum(inc, axis=0, keepdims=True)
    gsize = jnp.ceil(counts / _BLK) * _BLK
    off = jnp.concatenate([jnp.zeros((1, 1), jnp.float32), gsize[:, :-1]],
                          axis=1)
    j = 1
    while j < _E:
        off = off + jnp.concatenate(
            [jnp.zeros((1, j), jnp.float32), off[:, :-j]], axis=1)
        j *= 2

    rank1 = jnp.sum(c * oh1, axis=1, keepdims=True)
    rank2 = jnp.sum(c * oh2, axis=1, keepdims=True)
    off1 = jnp.sum(off * oh1, axis=1, keepdims=True)
    off2 = jnp.sum(off * oh2, axis=1, keepdims=True)
    pos1_ref[...] = (off1 + rank1).astype(jnp.int32)
    pos2_ref[...] = (off2 + rank2).astype(jnp.int32)

    ends = off + gsize
    jrow = lax.broadcasted_iota(jnp.int32, (_NTILES, _E), 0) * _BLK
    te = jnp.sum((jrow >= ends.astype(jnp.int32)).astype(jnp.int32), axis=1,
                 keepdims=True)
    valid = (te < _E).astype(jnp.int32)
    te_ref[...] = jnp.concatenate([jnp.minimum(te, _E - 1), valid], axis=1)


def _router(x, gate_w):
    return pl.pallas_call(
        _router_body,
        out_shape=(
            jax.ShapeDtypeStruct((_T, 1), jnp.int32),
            jax.ShapeDtypeStruct((_T, 1), jnp.int32),
            jax.ShapeDtypeStruct((_T, 1), jnp.float32),
            jax.ShapeDtypeStruct((_T, 1), jnp.float32),
            jax.ShapeDtypeStruct((_NTILES, 2), jnp.int32),
        ),
    )(x, gate_w)


def _expert_body(te_ref, xs_ref, wgu_ref, wd_ref, out_ref):
    i = pl.program_id(0)

    @pl.when(te_ref[i, 1] == 1)
    def _():
        x = xs_ref[...]
        gu = jnp.dot(x, wgu_ref[0], preferred_element_type=jnp.float32)
        g = gu[:, :_DFF]
        u = gu[:, _DFF:]
        h = g * jax.nn.sigmoid(g) * u
        out_ref[...] = jnp.dot(h, wd_ref[0], preferred_element_type=jnp.float32)

    @pl.when(te_ref[i, 1] == 0)
    def _():
        out_ref[...] = jnp.zeros_like(out_ref)


def _grouped_mlp(te, xs, w_gate_up, w_down):
    grid_spec = pltpu.PrefetchScalarGridSpec(
        num_scalar_prefetch=1,
        grid=(_NTILES,),
        in_specs=[
            pl.BlockSpec((_BLK, _D), lambda i, te: (i, 0)),
            pl.BlockSpec((1, _D, 2 * _DFF), lambda i, te: (te[i, 0], 0, 0)),
            pl.BlockSpec((1, _DFF, _D), lambda i, te: (te[i, 0], 0, 0)),
        ],
        out_specs=pl.BlockSpec((_BLK, _D), lambda i, te: (i, 0)),
    )
    return pl.pallas_call(
        _expert_body,
        grid_spec=grid_spec,
        out_shape=jax.ShapeDtypeStruct((_CAP, _D), jnp.float32),
        compiler_params=pltpu.CompilerParams(
            dimension_semantics=("arbitrary",)),
    )(te, xs, w_gate_up, w_down)


def _shared_body(x_ref, wgu_ref, wd_ref, out_ref):
    gu = jnp.dot(x_ref[...], wgu_ref[...], preferred_element_type=jnp.float32)
    g = gu[:, :_DFF]
    u = gu[:, _DFF:]
    h = g * jax.nn.sigmoid(g) * u
    out_ref[...] = jnp.dot(h, wd_ref[...], preferred_element_type=jnp.float32)


def _shared_mlp(x, ws_gate_up, ws_down):
    nblk = _T // _BLK
    return pl.pallas_call(
        _shared_body,
        grid=(nblk,),
        in_specs=[
            pl.BlockSpec((_BLK, _D), lambda i: (i, 0)),
            pl.BlockSpec((_D, 2 * _DFF), lambda i: (0, 0)),
            pl.BlockSpec((_DFF, _D), lambda i: (0, 0)),
        ],
        out_specs=pl.BlockSpec((_BLK, _D), lambda i: (i, 0)),
        out_shape=jax.ShapeDtypeStruct((_T, _D), jnp.float32),
    )(x, ws_gate_up, ws_down)


def kernel(hidden_states, gate_w, w_gate_up, w_down, ws_gate_up, ws_down):
    x = hidden_states
    pos1, pos2, w1, w2, te = _router(x, gate_w)
    pos1 = pos1[:, 0]
    pos2 = pos2[:, 0]

    xs = jnp.zeros((_CAP, _D), jnp.float32).at[pos1].set(x).at[pos2].set(x)

    hbuf = _grouped_mlp(te, xs, w_gate_up, w_down)
    shared = _shared_mlp(x, ws_gate_up, ws_down)

    out = shared + w1 * hbuf[pos1] + w2 * hbuf[pos2]
    return out
